# baseline (speedup 1.0000x reference)
import functools

import jax
import jax.numpy as jnp
from jax import lax
from jax.experimental import pallas as pl
from jax.experimental.pallas import tpu as pltpu
from jax.experimental.pallas import tpu_sc as plsc

NC = 2
NS = 16
L = 16
NW = NC * NS
B = 16384
D = 32
BPW = B // NW
G = BPW // L


def _gmf_body(users_hbm, items_hbm, ut_hbm, it_hbm, params_hbm, out_hbm,
              uidx_v, iidx_v, urows_v, irows_v, out_v, params_v,
              sem_u, sem_i):
    wid = lax.axis_index("s") * NC + lax.axis_index("c")
    base = wid * BPW

    pltpu.sync_copy(users_hbm.at[pl.ds(base, BPW)], uidx_v)
    pltpu.sync_copy(items_hbm.at[pl.ds(base, BPW)], iidx_v)
    cu = pltpu.async_copy(ut_hbm.at[uidx_v], urows_v, sem_u)
    ci = pltpu.async_copy(it_hbm.at[iidx_v], irows_v, sem_i)
    pltpu.sync_copy(params_hbm, params_v)
    cu.wait()
    ci.wait()

    w0 = params_v[pl.ds(0, L)]
    w1 = params_v[pl.ds(L, L)]
    bv = params_v[pl.ds(2 * L, L)]
    ws = [w0[d] for d in range(L)] + [w1[d] for d in range(L)]
    bias = bv[0]
    lanes = lax.iota(jnp.int32, L)

    def group(g, carry):
        row = g * L + lanes
        acc = jnp.zeros((L,), jnp.float32)
        for d in range(D):
            dcol = jnp.full((L,), d, jnp.int32)
            uu = plsc.load_gather(urows_v, [row, dcol])
            vv = plsc.load_gather(irows_v, [row, dcol])
            acc = acc + (uu * vv) * ws[d]
        z = acc + bias
        out_v[pl.ds(g * L, L)] = 1.0 / (1.0 + jnp.exp(-z))
        return carry

    lax.fori_loop(0, G, group, 0)
    pltpu.sync_copy(out_v, out_hbm.at[pl.ds(base, BPW)])


@jax.jit
def _gmf(users, items, user_table, item_table, params):
    mesh = plsc.VectorSubcoreMesh(core_axis_name="c", subcore_axis_name="s",
                                  num_cores=NC, num_subcores=NS)
    return pl.kernel(
        _gmf_body,
        out_type=jax.ShapeDtypeStruct((B,), jnp.float32),
        mesh=mesh,
        compiler_params=pltpu.CompilerParams(needs_layout_passes=False,
                                             use_tc_tiling_on_sc=False),
        scratch_types=[
            pltpu.VMEM((BPW,), jnp.int32),
            pltpu.VMEM((BPW,), jnp.int32),
            pltpu.VMEM((BPW, D), jnp.float32),
            pltpu.VMEM((BPW, D), jnp.float32),
            pltpu.VMEM((BPW,), jnp.float32),
            pltpu.VMEM((D + L,), jnp.float32),
            pltpu.SemaphoreType.DMA,
            pltpu.SemaphoreType.DMA,
        ],
    )(users, items, user_table, item_table, params)


def kernel(users, items, user_table, item_table, W, b):
    params = jnp.concatenate(
        [W.reshape(-1), b.reshape(-1),
         jnp.zeros((L - 1,), jnp.float32)]).astype(jnp.float32)
    return _gmf(users.astype(jnp.int32), items.astype(jnp.int32),
                user_table, item_table, params)


# speedup vs baseline: 3.4632x; 3.4632x over previous
import jax
import jax.numpy as jnp
from jax import lax
from jax.experimental import pallas as pl
from jax.experimental.pallas import tpu as pltpu
from jax.experimental.pallas import tpu_sc as plsc

NC = 2
NS = 16
L = 16
NW = NC * NS
B = 16384
D = 32
BPW = B // NW
SW = 16
NSW = BPW // SW
WIDS = 4
NSUB = SW // WIDS
BLK = 128


def _gmf_body(users_hbm, items_hbm, utT_hbm, itT_hbm, params_hbm, out_hbm,
              uidx_v, iidx_v, ublk_v, iblk_v, psums_v, out_v,
              params_v, sem_p, sem_a, sem_b):
    wid = lax.axis_index("s") * NC + lax.axis_index("c")
    base = wid * BPW

    pltpu.sync_copy(users_hbm.at[pl.ds(base, BPW)], uidx_v)
    pltpu.sync_copy(items_hbm.at[pl.ds(base, BPW)], iidx_v)
    cp = pltpu.async_copy(params_hbm, params_v, sem_p)
    cp.wait()

    w_lo = params_v[pl.ds(0, L)]
    w_hi = params_v[pl.ds(L, L)]
    bias = params_v[pl.ds(2 * L, L)][0]
    dlane = lax.iota(jnp.int32, L)

    sems = (sem_a, sem_b)

    def issue_sub(us, is_, sub):
        buf, sem = sub % 2, sems[sub % 2]
        for j in range(WIDS):
            ub = lax.shift_right_logical(us[sub * WIDS + j], 7)
            ib = lax.shift_right_logical(is_[sub * WIDS + j], 7)
            pltpu.async_copy(
                utT_hbm.at[:, pl.ds(pl.multiple_of(ub * BLK, BLK), BLK)],
                ublk_v.at[buf, j], sem)
            pltpu.async_copy(
                itT_hbm.at[:, pl.ds(pl.multiple_of(ib * BLK, BLK), BLK)],
                iblk_v.at[buf, j], sem)

    def drain_sub(sub):
        buf, sem = sub % 2, sems[sub % 2]
        for j in range(WIDS):
            pltpu.make_async_copy(utT_hbm.at[:, pl.ds(0, BLK)],
                                  ublk_v.at[buf, j], sem).wait()
            pltpu.make_async_copy(itT_hbm.at[:, pl.ds(0, BLK)],
                                  iblk_v.at[buf, j], sem).wait()

    def extract_sub(s, us, is_, sub):
        buf = sub % 2
        bufv = jnp.full((L,), buf, jnp.int32)
        for j in range(WIDS):
            k = s * SW + sub * WIDS + j
            jv = jnp.full((L,), j, jnp.int32)
            ucv = jnp.full((L,), lax.bitwise_and(us[sub * WIDS + j], 127), jnp.int32)
            icv = jnp.full((L,), lax.bitwise_and(is_[sub * WIDS + j], 127), jnp.int32)
            u_lo = plsc.load_gather(ublk_v, [bufv, jv, dlane, ucv])
            u_hi = plsc.load_gather(ublk_v, [bufv, jv, dlane + L, ucv])
            v_lo = plsc.load_gather(iblk_v, [bufv, jv, dlane, icv])
            v_hi = plsc.load_gather(iblk_v, [bufv, jv, dlane + L, icv])
            psum = (u_lo * v_lo) * w_lo + (u_hi * v_hi) * w_hi
            psums_v[pl.ds(k * L, L)] = psum

    def superwave(s, carry):
        uv = uidx_v[pl.ds(s * SW, SW)]
        iv = iidx_v[pl.ds(s * SW, SW)]
        us = [uv[j] for j in range(SW)]
        is_ = [iv[j] for j in range(SW)]
        issue_sub(us, is_, 0)
        issue_sub(us, is_, 1)
        for sub in range(NSUB):
            drain_sub(sub)
            extract_sub(s, us, is_, sub)
            if sub + 2 < NSUB:
                issue_sub(us, is_, sub + 2)
        return carry

    lax.fori_loop(0, NSW, superwave, 0)

    lane16 = dlane * L

    def reduce_group(g, carry):
        acc = jnp.zeros((L,), jnp.float32)
        for c in range(L):
            acc = acc + plsc.load_gather(psums_v, [g * (L * L) + lane16 + c])
        z = acc + bias
        out_v[pl.ds(g * L, L)] = 1.0 / (1.0 + jnp.exp(-z))
        return carry

    lax.fori_loop(0, BPW // L, reduce_group, 0)
    pltpu.sync_copy(out_v, out_hbm.at[pl.ds(base, BPW)])


@jax.jit
def _gmf(users, items, user_table_t, item_table_t, params):
    mesh = plsc.VectorSubcoreMesh(core_axis_name="c", subcore_axis_name="s",
                                  num_cores=NC, num_subcores=NS)
    return pl.kernel(
        _gmf_body,
        out_type=jax.ShapeDtypeStruct((B,), jnp.float32),
        mesh=mesh,
        compiler_params=pltpu.CompilerParams(needs_layout_passes=False),
        scratch_types=[
            pltpu.VMEM((BPW,), jnp.int32),
            pltpu.VMEM((BPW,), jnp.int32),
            pltpu.VMEM((2, WIDS, D, BLK), jnp.float32),
            pltpu.VMEM((2, WIDS, D, BLK), jnp.float32),
            pltpu.VMEM((BPW * L,), jnp.float32),
            pltpu.VMEM((BPW,), jnp.float32),
            pltpu.VMEM((D + L,), jnp.float32),
            pltpu.SemaphoreType.DMA,
            pltpu.SemaphoreType.DMA,
            pltpu.SemaphoreType.DMA,
        ],
    )(users, items, user_table_t, item_table_t, params)


def kernel(users, items, user_table, item_table, W, b):
    params = jnp.concatenate(
        [W.reshape(-1), b.reshape(-1),
         jnp.zeros((L - 1,), jnp.float32)]).astype(jnp.float32)
    return _gmf(users.astype(jnp.int32), items.astype(jnp.int32),
                user_table.T, item_table.T, params)
